# Initial kernel scaffold; baseline (speedup 1.0000x reference)
import jax
import jax.numpy as jnp
from jax.experimental import pallas as pl


def kernel(x, xdims, xsttype, edge_index, batch, emb, W_ih, W_hh, b_ih, b_hh, W1, b1, Wl1, bl1, Wr1, g1, be1, Wl2, bl2, Wr2, g2, be2, W2a, b2a, W2b, b2b, Wout, bout):
    raise NotImplementedError("write your pallas kernel here")


# speedup vs baseline: 4.0289x; 4.0289x over previous
import functools

import jax
import jax.numpy as jnp
from jax import lax
from jax.experimental import pallas as pl
from jax.experimental.pallas import tpu as pltpu
from jax.experimental.pallas import tpu_sc as plsc

N = 100000
E = 1600000
T = 12
F = 8
H = 48
EMB = 12
NSEG = 64
D = 48

NC, NS = 2, 16
NW = NC * NS
BLK = 128
NBUF = 6
NBLK_W = 396
NGRP = NBLK_W // NBUF
EP = NW * NBLK_W * BLK
ROWS_PER_TILE = 6256
ACC_ROWS = ROWS_PER_TILE * NS
ZR = 128

NB = 2000
GRID_N = N // NB


def _make_sc_agg(with_cnt: bool):
    npass = 3 + (1 if with_cnt else 0)
    mesh = plsc.VectorSubcoreMesh(core_axis_name="c", subcore_axis_name="s",
                                  num_cores=NC, num_subcores=NS)

    @functools.partial(
        pl.kernel,
        out_type=jax.ShapeDtypeStruct((NC, npass, ACC_ROWS, 16), jnp.float32),
        mesh=mesh,
        scratch_types=[
            pltpu.VMEM((2, NBUF, BLK), jnp.int32),
            pltpu.VMEM((2, NBUF, BLK), jnp.int32),
            pltpu.VMEM((NBUF, BLK, 16), jnp.float32),
            pltpu.VMEM((BLK, 16), jnp.float32),
            pltpu.VMEM((ZR, 16), jnp.float32),
            pltpu.VMEM_SHARED((ACC_ROWS, 16), jnp.float32),
            pltpu.SemaphoreType.DMA((NBUF,)),
            pltpu.SemaphoreType.DMA((NBUF,)),
            pltpu.SemaphoreType.DMA((2, 2)),
        ],
        compiler_params=pltpu.CompilerParams(use_tc_tiling_on_sc=False),
    )
    def sc_agg(src_hbm, dst_hbm, t0, t1, t2, out_hbm,
               sbuf, dbuf, rows, ones_v, zbuf, acc, gsem, ssem, isem):
        c = lax.axis_index("c")
        s = lax.axis_index("s")
        w = c * NS + s
        base = s * ROWS_PER_TILE

        @pl.loop(0, BLK)
        def _init_ones(i):
            ones_v[i, :] = jnp.ones((16,), jnp.float32)

        @pl.loop(0, ZR)
        def _init_zeros(i):
            zbuf[i, :] = jnp.zeros((16,), jnp.float32)

        def zero_acc():
            nz = ROWS_PER_TILE // ZR
            rem = ROWS_PER_TILE - nz * ZR
            for ci in range(nz):
                pltpu.sync_copy(zbuf, acc.at[pl.ds(base + ci * ZR, ZR)])
            pltpu.sync_copy(zbuf.at[pl.ds(0, rem)],
                            acc.at[pl.ds(base + nz * ZR, rem)])

        def stage_idx(g, q, need_src):
            grp = pl.ds(g * NBUF, NBUF)
            if need_src:
                pltpu.async_copy(src_hbm.at[w, grp], sbuf.at[q], isem.at[q, 0])
            pltpu.async_copy(dst_hbm.at[w, grp], dbuf.at[q], isem.at[q, 1])

        def wait_idx(g, q, need_src):
            grp = pl.ds(g * NBUF, NBUF)
            if need_src:
                pltpu.make_async_copy(src_hbm.at[w, grp], sbuf.at[q],
                                      isem.at[q, 0]).wait()
            pltpu.make_async_copy(dst_hbm.at[w, grp], dbuf.at[q],
                                  isem.at[q, 1]).wait()

        def drain(p):
            plsc.subcore_barrier()
            pltpu.sync_copy(acc.at[pl.ds(base, ROWS_PER_TILE)],
                            out_hbm.at[c, p, pl.ds(base, ROWS_PER_TILE)])
            plsc.subcore_barrier()

        def feature_pass(table, p):
            zero_acc()
            plsc.subcore_barrier()
            stage_idx(0, 0, True)
            wait_idx(0, 0, True)
            stage_idx(1, 1, True)
            for b in range(NBUF):
                pltpu.async_copy(table.at[sbuf.at[0, b]], rows.at[b],
                                 gsem.at[b])

            @pl.loop(0, NGRP)
            def _grp(g):
                pb = lax.rem(g, 2)
                qb = lax.rem(g + 1, 2)
                for b in range(NBUF):
                    pltpu.make_async_copy(table.at[sbuf.at[pb, b]],
                                          rows.at[b], gsem.at[b]).wait()
                    pltpu.async_copy(rows.at[b], acc.at[dbuf.at[pb, b]],
                                     ssem.at[b], add=True)
                for b in range(NBUF):
                    pltpu.make_async_copy(rows.at[b], acc.at[dbuf.at[pb, b]],
                                          ssem.at[b]).wait()

                @pl.when(g + 2 < NGRP)
                def _():
                    stage_idx(g + 2, pb, True)

                @pl.when(g + 1 < NGRP)
                def _():
                    wait_idx(g + 1, qb, True)
                    for b in range(NBUF):
                        pltpu.async_copy(table.at[sbuf.at[qb, b]],
                                         rows.at[b], gsem.at[b])

            drain(p)

        def cnt_pass(p):
            zero_acc()
            plsc.subcore_barrier()
            stage_idx(0, 0, False)
            wait_idx(0, 0, False)
            stage_idx(1, 1, False)

            @pl.loop(0, NGRP)
            def _grp(g):
                pb = lax.rem(g, 2)
                qb = lax.rem(g + 1, 2)
                for b in range(NBUF):
                    pltpu.async_copy(ones_v, acc.at[dbuf.at[pb, b]],
                                     ssem.at[b], add=True)
                for b in range(NBUF):
                    pltpu.make_async_copy(ones_v, acc.at[dbuf.at[pb, b]],
                                          ssem.at[b]).wait()

                @pl.when(g + 2 < NGRP)
                def _():
                    stage_idx(g + 2, pb, False)

                @pl.when(g + 1 < NGRP)
                def _():
                    wait_idx(g + 1, qb, False)

            drain(p)

        feature_pass(t0, 0)
        feature_pass(t1, 1)
        feature_pass(t2, 2)
        if with_cnt:
            cnt_pass(3)

    return sc_agg


@functools.lru_cache(maxsize=None)
def _get_sc_agg(with_cnt: bool):
    return _make_sc_agg(with_cnt)


def _encoder_body(x_ref, xd_ref, xst_ref, emb_ref,
                  wir, wiz, win, whr, whz, whn,
                  bir, biz, bin_, bhr, bhz, bhn,
                  w1h, w1d, w1s, b1_ref,
                  h48, o0, o1, o2):
    f32 = jnp.float32
    xs = xst_ref[0, 0, :]
    oh = (xs[:, None] == lax.broadcasted_iota(jnp.int32, (1, 256), 1)).astype(f32)
    st = jnp.dot(oh, emb_ref[...], preferred_element_type=f32)

    h = jnp.zeros((NB, H), f32)
    for t in range(T):
        xt = x_ref[:, t, :]
        i_r = jnp.dot(xt, wir[...], preferred_element_type=f32) + bir[...]
        i_z = jnp.dot(xt, wiz[...], preferred_element_type=f32) + biz[...]
        i_n = jnp.dot(xt, win[...], preferred_element_type=f32) + bin_[...]
        h_r = jnp.dot(h, whr[...], preferred_element_type=f32) + bhr[...]
        h_z = jnp.dot(h, whz[...], preferred_element_type=f32) + bhz[...]
        h_n = jnp.dot(h, whn[...], preferred_element_type=f32) + bhn[...]
        r = jax.nn.sigmoid(i_r + h_r)
        z = jax.nn.sigmoid(i_z + h_z)
        n = jnp.tanh(i_n + r * h_n)
        h = (1.0 - z) * n + z * h

    h1 = (jnp.dot(h, w1h[...], preferred_element_type=f32)
          + jnp.dot(xd_ref[...], w1d[...], preferred_element_type=f32)
          + jnp.dot(st, w1s[...], preferred_element_type=f32)
          + b1_ref[...])
    h1 = jnp.maximum(h1, 0.0)
    h48[...] = h1
    o0[...] = h1[:, 0:16]
    o1[...] = h1[:, 16:32]
    o2[...] = h1[:, 32:48]


def _encoder(x, xdims, xst3, emb, gw, w1h, w1d, w1s, b1):
    f32 = jnp.float32
    full = lambda shp: pl.BlockSpec(shp, lambda i: (0,) * len(shp))
    in_specs = [
        pl.BlockSpec((NB, T, F), lambda i: (i, 0, 0)),
        pl.BlockSpec((NB, 2), lambda i: (i, 0)),
        pl.BlockSpec((1, 1, NB), lambda i: (i, 0, 0)),
        full((256, EMB)),
    ]
    in_specs += [full(w.shape) for w in gw]
    in_specs += [full((H, H)), full((2, H)), full((EMB, H)), full((1, H))]
    out_shape = [
        jax.ShapeDtypeStruct((N, H), f32),
        jax.ShapeDtypeStruct((N, 16), f32),
        jax.ShapeDtypeStruct((N, 16), f32),
        jax.ShapeDtypeStruct((N, 16), f32),
    ]
    out_specs = [
        pl.BlockSpec((NB, H), lambda i: (i, 0)),
        pl.BlockSpec((NB, 16), lambda i: (i, 0)),
        pl.BlockSpec((NB, 16), lambda i: (i, 0)),
        pl.BlockSpec((NB, 16), lambda i: (i, 0)),
    ]
    return pl.pallas_call(
        _encoder_body, grid=(GRID_N,), in_specs=in_specs,
        out_specs=out_specs, out_shape=out_shape,
    )(x, xdims, xst3, emb, *gw, w1h, w1d, w1s, b1)


def _transform_body(first, h_ref, a0, a1, a2, p0, p1, p2, c0, c1,
                    wl0, wl1, wl2, bl_ref, wr, g_ref, be_ref, *out_refs):
    f32 = jnp.float32
    cnt = (c0[...] + c1[...])
    if first:
        cnt = cnt[:, 0:1]
    inv = 1.0 / jnp.maximum(cnt, 1.0)
    pre = (jnp.dot((a0[...] + p0[...]) * inv, wl0[...], preferred_element_type=f32)
           + jnp.dot((a1[...] + p1[...]) * inv, wl1[...], preferred_element_type=f32)
           + jnp.dot((a2[...] + p2[...]) * inv, wl2[...], preferred_element_type=f32)
           + bl_ref[...]
           + jnp.dot(h_ref[...], wr[...], preferred_element_type=f32))
    mu = jnp.mean(pre, axis=-1, keepdims=True)
    var = jnp.mean((pre - mu) ** 2, axis=-1, keepdims=True)
    y = (pre - mu) * lax.rsqrt(var + 1e-5) * g_ref[...] + be_ref[...]
    y = jnp.maximum(y, 0.0)
    out_refs[0][...] = y
    if first:
        out_refs[1][...] = y[:, 0:16]
        out_refs[2][...] = y[:, 16:32]
        out_refs[3][...] = y[:, 32:48]
        out_refs[4][...] = cnt


def _transform(first, h, parts0, parts1, cnt0, cnt1, wl, bl, wr, g, be):
    f32 = jnp.float32
    full = lambda shp: pl.BlockSpec(shp, lambda i: (0,) * len(shp))
    nb16 = pl.BlockSpec((NB, 16), lambda i: (i, 0))
    in_specs = [pl.BlockSpec((NB, H), lambda i: (i, 0))]
    in_specs += [nb16] * 6
    if first:
        in_specs += [nb16] * 2
    else:
        in_specs += [pl.BlockSpec((NB, 1), lambda i: (i, 0))] * 2
    in_specs += [full((16, H))] * 3 + [full((1, H)), full((H, H)),
                                       full((1, H)), full((1, H))]
    out_shape = [jax.ShapeDtypeStruct((N, H), f32)]
    out_specs = [pl.BlockSpec((NB, H), lambda i: (i, 0))]
    if first:
        out_shape += [jax.ShapeDtypeStruct((N, 16), f32)] * 3
        out_shape += [jax.ShapeDtypeStruct((N, 1), f32)]
        out_specs += [nb16] * 3 + [pl.BlockSpec((NB, 1), lambda i: (i, 0))]
    wl0 = wl[:, 0:16].T
    wl1 = wl[:, 16:32].T
    wl2 = wl[:, 32:48].T
    return pl.pallas_call(
        functools.partial(_transform_body, first),
        grid=(GRID_N,), in_specs=in_specs, out_specs=out_specs,
        out_shape=out_shape,
    )(h, *parts0, *parts1, cnt0, cnt1, wl0, wl1, wl2,
      bl.reshape(1, H), wr.T, g.reshape(1, H), be.reshape(1, H))


def _pool_body(h_ref, b_ref, sums, cnts, maxs):
    f32 = jnp.float32
    i = pl.program_id(0)

    @pl.when(i == 0)
    def _():
        sums[...] = jnp.zeros((NSEG, H), f32)
        cnts[...] = jnp.zeros((NSEG, 1), f32)
        maxs[...] = jnp.full((NSEG, H), -jnp.inf, f32)

    b = b_ref[0, 0, :]
    h = h_ref[...]
    oh = (b[:, None] == lax.broadcasted_iota(jnp.int32, (1, NSEG), 1)).astype(f32)
    sums[...] += lax.dot_general(oh, h, (((0,), (0,)), ((), ())),
                                 preferred_element_type=f32)
    cnts[...] += lax.dot_general(oh, jnp.ones((NB, 1), f32),
                                 (((0,), (0,)), ((), ())),
                                 preferred_element_type=f32)
    for s in range(NSEG):
        m = jnp.max(jnp.where(b[:, None] == s, h, -jnp.inf), axis=0)
        maxs[s:s + 1, :] = jnp.maximum(maxs[s:s + 1, :], m[None, :])


def _pool(h, batch3):
    f32 = jnp.float32
    full0 = lambda shp: pl.BlockSpec(shp, lambda i: (0,) * len(shp))
    return pl.pallas_call(
        _pool_body, grid=(GRID_N,),
        in_specs=[pl.BlockSpec((NB, H), lambda i: (i, 0)),
                  pl.BlockSpec((1, 1, NB), lambda i: (i, 0, 0))],
        out_specs=[full0((NSEG, H)), full0((NSEG, 1)), full0((NSEG, H))],
        out_shape=[jax.ShapeDtypeStruct((NSEG, H), f32),
                   jax.ShapeDtypeStruct((NSEG, 1), f32),
                   jax.ShapeDtypeStruct((NSEG, H), f32)],
    )(h, batch3)


def _head_body(sums, cnts, maxs, w2am, w2ax, b2a, w2b, b2b, wo, bo, out):
    f32 = jnp.float32
    gmean = sums[...] / jnp.maximum(cnts[...], 1.0)
    g = (jnp.dot(gmean, w2am[...], preferred_element_type=f32)
         + jnp.dot(maxs[...], w2ax[...], preferred_element_type=f32)
         + b2a[...])
    g = jnp.maximum(g, 0.0)
    g = jnp.maximum(jnp.dot(g, w2b[...], preferred_element_type=f32) + b2b[...], 0.0)
    out[...] = jnp.dot(g, wo[...], preferred_element_type=f32) + bo[...]


def _head(sums, cnts, maxs, W2a, b2a, W2b, b2b, Wout, bout):
    f32 = jnp.float32
    return pl.pallas_call(
        _head_body,
        out_shape=jax.ShapeDtypeStruct((NSEG, 1), f32),
    )(sums, cnts, maxs, W2a[:, :D].T, W2a[:, D:].T, b2a.reshape(1, 50),
      W2b.T, b2b.reshape(1, 50), Wout.T, bout.reshape(1, 1))


def kernel(x, xdims, xsttype, edge_index, batch, emb, W_ih, W_hh, b_ih, b_hh,
           W1, b1, Wl1, bl1, Wr1, g1, be1, Wl2, bl2, Wr2, g2, be2,
           W2a, b2a, W2b, b2b, Wout, bout):
    f32 = jnp.float32
    gw = [W_ih[0:H].T, W_ih[H:2 * H].T, W_ih[2 * H:].T,
          W_hh[0:H].T, W_hh[H:2 * H].T, W_hh[2 * H:].T,
          b_ih[0:H].reshape(1, H), b_ih[H:2 * H].reshape(1, H),
          b_ih[2 * H:].reshape(1, H),
          b_hh[0:H].reshape(1, H), b_hh[H:2 * H].reshape(1, H),
          b_hh[2 * H:].reshape(1, H)]
    w1h = W1[:, 0:H].T
    w1d = W1[:, H:H + 2].T
    w1s = W1[:, H + 2:].T
    xst3 = xsttype.astype(jnp.int32).reshape(GRID_N, 1, NB)
    batch3 = batch.astype(jnp.int32).reshape(GRID_N, 1, NB)

    src = edge_index[0].astype(jnp.int32)
    dst = edge_index[1].astype(jnp.int32)
    pad = EP - E
    src3 = jnp.concatenate([src, jnp.zeros((pad,), jnp.int32)]
                           ).reshape(NW, NBLK_W, BLK)
    dst3 = jnp.concatenate([dst, jnp.full((pad,), N, jnp.int32)]
                           ).reshape(NW, NBLK_W, BLK)

    h48, t0, t1, t2 = _encoder(x, xdims, xst3, emb, gw,
                               w1h, w1d, w1s, b1.reshape(1, H))

    p1 = _get_sc_agg(True)(src3, dst3, t0, t1, t2)
    outs = _transform(True, h48,
                      [p1[0, 0, :N], p1[0, 1, :N], p1[0, 2, :N]],
                      [p1[1, 0, :N], p1[1, 1, :N], p1[1, 2, :N]],
                      p1[0, 3, :N], p1[1, 3, :N],
                      Wl1, bl1, Wr1, g1, be1)
    h2, u0, u1, u2, cnt = outs

    p2 = _get_sc_agg(False)(src3, dst3, u0, u1, u2)
    h3 = _transform(False, h2,
                    [p2[0, 0, :N], p2[0, 1, :N], p2[0, 2, :N]],
                    [p2[1, 0, :N], p2[1, 1, :N], p2[1, 2, :N]],
                    cnt, jnp.zeros_like(cnt),
                    Wl2, bl2, Wr2, g2, be2)[0]

    sums, cnts, maxs = _pool(h3, batch3)
    return _head(sums, cnts, maxs, W2a, b2a, W2b, b2b, Wout, bout)


# speedup vs baseline: 4.9516x; 1.2290x over previous
import functools

import jax
import jax.numpy as jnp
from jax import lax
from jax.experimental import pallas as pl
from jax.experimental.pallas import tpu as pltpu
from jax.experimental.pallas import tpu_sc as plsc

N = 100000
E = 1600000
T = 12
F = 8
H = 48
EMB = 12
NSEG = 64
D = 48

NC, NS = 2, 16
NW = NC * NS
BLK = 128
NBUF = 6
NBLK_W = 396
NGRP = NBLK_W // NBUF
EP = NW * NBLK_W * BLK
ROWS_PER_TILE = 6256
ACC_ROWS = ROWS_PER_TILE * NS
ZR = 128

NB = 2000
GRID_N = N // NB
ENB = 1000
EGRID = N // ENB


def _make_sc_agg(with_cnt: bool):
    npass = 3 + (1 if with_cnt else 0)
    mesh = plsc.VectorSubcoreMesh(core_axis_name="c", subcore_axis_name="s",
                                  num_cores=NC, num_subcores=NS)

    @functools.partial(
        pl.kernel,
        out_type=jax.ShapeDtypeStruct((NC, npass, ACC_ROWS, 16), jnp.float32),
        mesh=mesh,
        scratch_types=[
            pltpu.VMEM((2, NBUF, BLK), jnp.int32),
            pltpu.VMEM((2, NBUF, BLK), jnp.int32),
            pltpu.VMEM((NBUF, BLK, 16), jnp.float32),
            pltpu.VMEM((BLK, 16), jnp.float32),
            pltpu.VMEM((ZR, 16), jnp.float32),
            pltpu.VMEM_SHARED((ACC_ROWS, 16), jnp.float32),
            pltpu.SemaphoreType.DMA((NBUF,)),
            pltpu.SemaphoreType.DMA((NBUF,)),
            pltpu.SemaphoreType.DMA((2, 2)),
        ],
        compiler_params=pltpu.CompilerParams(use_tc_tiling_on_sc=False),
    )
    def sc_agg(src_hbm, dst_hbm, t0, t1, t2, out_hbm,
               sbuf, dbuf, rows, ones_v, zbuf, acc, gsem, ssem, isem):
        c = lax.axis_index("c")
        s = lax.axis_index("s")
        w = c * NS + s
        base = s * ROWS_PER_TILE

        @pl.loop(0, BLK)
        def _init_ones(i):
            ones_v[i, :] = jnp.ones((16,), jnp.float32)

        @pl.loop(0, ZR)
        def _init_zeros(i):
            zbuf[i, :] = jnp.zeros((16,), jnp.float32)

        def zero_acc():
            nz = ROWS_PER_TILE // ZR
            rem = ROWS_PER_TILE - nz * ZR
            for ci in range(nz):
                pltpu.sync_copy(zbuf, acc.at[pl.ds(base + ci * ZR, ZR)])
            pltpu.sync_copy(zbuf.at[pl.ds(0, rem)],
                            acc.at[pl.ds(base + nz * ZR, rem)])

        def stage_idx(g, q, need_src):
            grp = pl.ds(g * NBUF, NBUF)
            if need_src:
                pltpu.async_copy(src_hbm.at[w, grp], sbuf.at[q], isem.at[q, 0])
            pltpu.async_copy(dst_hbm.at[w, grp], dbuf.at[q], isem.at[q, 1])

        def wait_idx(g, q, need_src):
            grp = pl.ds(g * NBUF, NBUF)
            if need_src:
                pltpu.make_async_copy(src_hbm.at[w, grp], sbuf.at[q],
                                      isem.at[q, 0]).wait()
            pltpu.make_async_copy(dst_hbm.at[w, grp], dbuf.at[q],
                                  isem.at[q, 1]).wait()

        def drain(p):
            plsc.subcore_barrier()
            pltpu.sync_copy(acc.at[pl.ds(base, ROWS_PER_TILE)],
                            out_hbm.at[c, p, pl.ds(base, ROWS_PER_TILE)])
            plsc.subcore_barrier()

        def feature_pass(table, p):
            zero_acc()
            plsc.subcore_barrier()
            stage_idx(0, 0, True)
            wait_idx(0, 0, True)
            stage_idx(1, 1, True)
            for b in range(NBUF):
                pltpu.async_copy(table.at[sbuf.at[0, b]], rows.at[b],
                                 gsem.at[b])

            @pl.loop(0, NGRP)
            def _grp(g):
                pb = lax.rem(g, 2)
                qb = lax.rem(g + 1, 2)
                for b in range(NBUF):
                    pltpu.make_async_copy(table.at[sbuf.at[pb, b]],
                                          rows.at[b], gsem.at[b]).wait()
                    pltpu.async_copy(rows.at[b], acc.at[dbuf.at[pb, b]],
                                     ssem.at[b], add=True)
                for b in range(NBUF):
                    pltpu.make_async_copy(rows.at[b], acc.at[dbuf.at[pb, b]],
                                          ssem.at[b]).wait()

                @pl.when(g + 2 < NGRP)
                def _():
                    stage_idx(g + 2, pb, True)

                @pl.when(g + 1 < NGRP)
                def _():
                    wait_idx(g + 1, qb, True)
                    for b in range(NBUF):
                        pltpu.async_copy(table.at[sbuf.at[qb, b]],
                                         rows.at[b], gsem.at[b])

            drain(p)

        def cnt_pass(p):
            zero_acc()
            plsc.subcore_barrier()
            stage_idx(0, 0, False)
            wait_idx(0, 0, False)
            stage_idx(1, 1, False)

            @pl.loop(0, NGRP)
            def _grp(g):
                pb = lax.rem(g, 2)
                qb = lax.rem(g + 1, 2)
                for b in range(NBUF):
                    pltpu.async_copy(ones_v, acc.at[dbuf.at[pb, b]],
                                     ssem.at[b], add=True)
                for b in range(NBUF):
                    pltpu.make_async_copy(ones_v, acc.at[dbuf.at[pb, b]],
                                          ssem.at[b]).wait()

                @pl.when(g + 2 < NGRP)
                def _():
                    stage_idx(g + 2, pb, False)

                @pl.when(g + 1 < NGRP)
                def _():
                    wait_idx(g + 1, qb, False)

            drain(p)

        feature_pass(t0, 0)
        feature_pass(t1, 1)
        feature_pass(t2, 2)
        if with_cnt:
            cnt_pass(3)

    return sc_agg


@functools.lru_cache(maxsize=None)
def _get_sc_agg(with_cnt: bool):
    return _make_sc_agg(with_cnt)


def _encoder_body(x_ref, xd_ref, xst_ref, emb_ref,
                  wir, wiz, win, whr, whz, whn,
                  bir, biz, bin_, bhr, bhz, bhn,
                  w1h, w1d, w1s, b1_ref,
                  h48, o0, o1, o2):
    f32 = jnp.float32
    xs = xst_ref[0, 0, :]
    oh = (xs[:, None] == lax.broadcasted_iota(jnp.int32, (1, 256), 1)
          ).astype(f32)
    st = jnp.dot(oh, emb_ref[...], preferred_element_type=f32)

    bf16 = jnp.bfloat16
    xall = x_ref[...].reshape(T * ENB, F).astype(bf16)
    gr = jnp.dot(xall, wir[...].astype(bf16), preferred_element_type=f32) + bir[...]
    gz = jnp.dot(xall, wiz[...].astype(bf16), preferred_element_type=f32) + biz[...]
    gn = jnp.dot(xall, win[...].astype(bf16), preferred_element_type=f32) + bin_[...]

    whr_b = whr[...].astype(bf16)
    whz_b = whz[...].astype(bf16)
    whn_b = whn[...].astype(bf16)
    h = jnp.zeros((ENB, H), f32)
    for t in range(T):
        row = slice(t * ENB, (t + 1) * ENB)
        hb = h.astype(bf16)
        h_r = jnp.dot(hb, whr_b, preferred_element_type=f32)
        h_z = jnp.dot(hb, whz_b, preferred_element_type=f32)
        h_n = jnp.dot(hb, whn_b, preferred_element_type=f32) + bhn[...]
        r = 0.5 * jnp.tanh(gr[row] + h_r) + 0.5
        z = 0.5 * jnp.tanh(gz[row] + h_z) + 0.5
        n = jnp.tanh(gn[row] + r * h_n)
        h = n + z * (h - n)

    h1 = (jnp.dot(h.astype(bf16), w1h[...].astype(bf16),
                  preferred_element_type=f32)
          + jnp.dot(xd_ref[...].astype(bf16), w1d[...].astype(bf16),
                    preferred_element_type=f32)
          + jnp.dot(st.astype(bf16), w1s[...].astype(bf16),
                    preferred_element_type=f32)
          + b1_ref[...])
    h1 = jnp.maximum(h1, 0.0)
    h48[...] = h1
    o0[...] = h1[:, 0:16]
    o1[...] = h1[:, 16:32]
    o2[...] = h1[:, 32:48]


def _encoder(x, xdims, xst3, emb, gw, w1h, w1d, w1s, b1):
    f32 = jnp.float32
    full = lambda shp: pl.BlockSpec(shp, lambda i: (0,) * len(shp))
    in_specs = [
        pl.BlockSpec((T, ENB, F), lambda i: (0, i, 0)),
        pl.BlockSpec((ENB, 2), lambda i: (i, 0)),
        pl.BlockSpec((1, 1, ENB), lambda i: (i, 0, 0)),
        full((256, EMB)),
    ]
    in_specs += [full(w.shape) for w in gw]
    in_specs += [full((H, H)), full((2, H)), full((EMB, H)), full((1, H))]
    out_shape = [
        jax.ShapeDtypeStruct((N, H), f32),
        jax.ShapeDtypeStruct((N, 16), f32),
        jax.ShapeDtypeStruct((N, 16), f32),
        jax.ShapeDtypeStruct((N, 16), f32),
    ]
    out_specs = [
        pl.BlockSpec((ENB, H), lambda i: (i, 0)),
        pl.BlockSpec((ENB, 16), lambda i: (i, 0)),
        pl.BlockSpec((ENB, 16), lambda i: (i, 0)),
        pl.BlockSpec((ENB, 16), lambda i: (i, 0)),
    ]
    return pl.pallas_call(
        _encoder_body, grid=(EGRID,), in_specs=in_specs,
        out_specs=out_specs, out_shape=out_shape,
    )(x, xdims, xst3, emb, *gw, w1h, w1d, w1s, b1)


def _transform_body(first, h_ref, a0, a1, a2, p0, p1, p2, *rest):
    f32 = jnp.float32
    if first:
        c0, c1, wl0, wl1, wl2, bl_ref, wr, g_ref, be_ref = rest[:9]
        out_refs = rest[9:]
        cnt = (c0[0, 0] + c1[0, 0])[:, 0:1]
    else:
        c0, wl0, wl1, wl2, bl_ref, wr, g_ref, be_ref = rest[:8]
        out_refs = rest[8:]
        cnt = c0[...]
    inv = 1.0 / jnp.maximum(cnt, 1.0)
    pre = (jnp.dot((a0[0, 0] + p0[0, 0]) * inv, wl0[...],
                   preferred_element_type=f32)
           + jnp.dot((a1[0, 0] + p1[0, 0]) * inv, wl1[...],
                     preferred_element_type=f32)
           + jnp.dot((a2[0, 0] + p2[0, 0]) * inv, wl2[...],
                     preferred_element_type=f32)
           + bl_ref[...]
           + jnp.dot(h_ref[...], wr[...], preferred_element_type=f32))
    mu = jnp.mean(pre, axis=-1, keepdims=True)
    var = jnp.mean((pre - mu) ** 2, axis=-1, keepdims=True)
    y = (pre - mu) * lax.rsqrt(var + 1e-5) * g_ref[...] + be_ref[...]
    y = jnp.maximum(y, 0.0)
    out_refs[0][...] = y
    if first:
        out_refs[1][...] = y[:, 0:16]
        out_refs[2][...] = y[:, 16:32]
        out_refs[3][...] = y[:, 32:48]
        out_refs[4][...] = cnt


def _part_spec(c, k):
    return pl.BlockSpec((1, 1, NB, 16),
                        lambda i, _c=c, _k=k: (_c, _k, i, 0))


def _transform(first, h, parts, cnt, wl, bl, wr, g, be):
    f32 = jnp.float32
    full = lambda shp: pl.BlockSpec(shp, lambda i: (0,) * len(shp))
    nb16 = pl.BlockSpec((NB, 16), lambda i: (i, 0))
    in_specs = [pl.BlockSpec((NB, H), lambda i: (i, 0))]
    in_specs += [_part_spec(0, 0), _part_spec(0, 1), _part_spec(0, 2),
                 _part_spec(1, 0), _part_spec(1, 1), _part_spec(1, 2)]
    if first:
        in_specs += [_part_spec(0, 3), _part_spec(1, 3)]
        cnt_args = (parts, parts)
    else:
        in_specs += [pl.BlockSpec((NB, 1), lambda i: (i, 0))]
        cnt_args = (cnt,)
    in_specs += [full((16, H))] * 3 + [full((1, H)), full((H, H)),
                                       full((1, H)), full((1, H))]
    out_shape = [jax.ShapeDtypeStruct((N, H), f32)]
    out_specs = [pl.BlockSpec((NB, H), lambda i: (i, 0))]
    if first:
        out_shape += [jax.ShapeDtypeStruct((N, 16), f32)] * 3
        out_shape += [jax.ShapeDtypeStruct((N, 1), f32)]
        out_specs += [nb16] * 3 + [pl.BlockSpec((NB, 1), lambda i: (i, 0))]
    wl0 = wl[:, 0:16].T
    wl1 = wl[:, 16:32].T
    wl2 = wl[:, 32:48].T
    return pl.pallas_call(
        functools.partial(_transform_body, first),
        grid=(GRID_N,), in_specs=in_specs, out_specs=out_specs,
        out_shape=out_shape,
    )(h, parts, parts, parts, parts, parts, parts, *cnt_args,
      wl0, wl1, wl2, bl.reshape(1, H), wr.T, g.reshape(1, H),
      be.reshape(1, H))


def _pool_body(h_ref, b_ref, sums, cnts, maxs):
    f32 = jnp.float32
    i = pl.program_id(0)

    @pl.when(i == 0)
    def _():
        sums[...] = jnp.zeros((NSEG, H), f32)
        cnts[...] = jnp.zeros((NSEG, 1), f32)
        maxs[...] = jnp.full((NSEG, H), -jnp.inf, f32)

    b = b_ref[0, 0, :]
    h = h_ref[...]
    oh = (b[:, None] == lax.broadcasted_iota(jnp.int32, (1, NSEG), 1)).astype(f32)
    sums[...] += lax.dot_general(oh, h, (((0,), (0,)), ((), ())),
                                 preferred_element_type=f32)
    cnts[...] += lax.dot_general(oh, jnp.ones((NB, 1), f32),
                                 (((0,), (0,)), ((), ())),
                                 preferred_element_type=f32)
    for s in range(NSEG):
        m = jnp.max(jnp.where(b[:, None] == s, h, -jnp.inf), axis=0)
        maxs[s:s + 1, :] = jnp.maximum(maxs[s:s + 1, :], m[None, :])


def _pool(h, batch3):
    f32 = jnp.float32
    full0 = lambda shp: pl.BlockSpec(shp, lambda i: (0,) * len(shp))
    return pl.pallas_call(
        _pool_body, grid=(GRID_N,),
        in_specs=[pl.BlockSpec((NB, H), lambda i: (i, 0)),
                  pl.BlockSpec((1, 1, NB), lambda i: (i, 0, 0))],
        out_specs=[full0((NSEG, H)), full0((NSEG, 1)), full0((NSEG, H))],
        out_shape=[jax.ShapeDtypeStruct((NSEG, H), f32),
                   jax.ShapeDtypeStruct((NSEG, 1), f32),
                   jax.ShapeDtypeStruct((NSEG, H), f32)],
    )(h, batch3)


def _head_body(sums, cnts, maxs, w2am, w2ax, b2a, w2b, b2b, wo, bo, out):
    f32 = jnp.float32
    gmean = sums[...] / jnp.maximum(cnts[...], 1.0)
    g = (jnp.dot(gmean, w2am[...], preferred_element_type=f32)
         + jnp.dot(maxs[...], w2ax[...], preferred_element_type=f32)
         + b2a[...])
    g = jnp.maximum(g, 0.0)
    g = jnp.maximum(jnp.dot(g, w2b[...], preferred_element_type=f32) + b2b[...], 0.0)
    out[...] = jnp.dot(g, wo[...], preferred_element_type=f32) + bo[...]


def _head(sums, cnts, maxs, W2a, b2a, W2b, b2b, Wout, bout):
    f32 = jnp.float32
    return pl.pallas_call(
        _head_body,
        out_shape=jax.ShapeDtypeStruct((NSEG, 1), f32),
    )(sums, cnts, maxs, W2a[:, :D].T, W2a[:, D:].T, b2a.reshape(1, 50),
      W2b.T, b2b.reshape(1, 50), Wout.T, bout.reshape(1, 1))


def kernel(x, xdims, xsttype, edge_index, batch, emb, W_ih, W_hh, b_ih, b_hh,
           W1, b1, Wl1, bl1, Wr1, g1, be1, Wl2, bl2, Wr2, g2, be2,
           W2a, b2a, W2b, b2b, Wout, bout):
    f32 = jnp.float32
    gw = [0.5 * W_ih[0:H].T, 0.5 * W_ih[H:2 * H].T, W_ih[2 * H:].T,
          0.5 * W_hh[0:H].T, 0.5 * W_hh[H:2 * H].T, W_hh[2 * H:].T,
          (0.5 * (b_ih[0:H] + b_hh[0:H])).reshape(1, H),
          (0.5 * (b_ih[H:2 * H] + b_hh[H:2 * H])).reshape(1, H),
          b_ih[2 * H:].reshape(1, H),
          jnp.zeros((1, H), jnp.float32), jnp.zeros((1, H), jnp.float32),
          b_hh[2 * H:].reshape(1, H)]
    w1h = W1[:, 0:H].T
    w1d = W1[:, H:H + 2].T
    w1s = W1[:, H + 2:].T
    xst3 = xsttype.astype(jnp.int32).reshape(EGRID, 1, ENB)
    batch3 = batch.astype(jnp.int32).reshape(GRID_N, 1, NB)

    src = edge_index[0].astype(jnp.int32)
    dst = edge_index[1].astype(jnp.int32)
    pad = EP - E
    src3 = jnp.concatenate([src, jnp.zeros((pad,), jnp.int32)]
                           ).reshape(NW, NBLK_W, BLK)
    dst3 = jnp.concatenate([dst, jnp.full((pad,), N, jnp.int32)]
                           ).reshape(NW, NBLK_W, BLK)

    xT = jnp.transpose(x, (1, 0, 2))
    h48, t0, t1, t2 = _encoder(xT, xdims, xst3, emb, gw,
                               w1h, w1d, w1s, b1.reshape(1, H))

    p1 = _get_sc_agg(True)(src3, dst3, t0, t1, t2)
    h2, u0, u1, u2, cnt = _transform(True, h48, p1, None,
                                     Wl1, bl1, Wr1, g1, be1)

    p2 = _get_sc_agg(False)(src3, dst3, u0, u1, u2)
    h3 = _transform(False, h2, p2, cnt, Wl2, bl2, Wr2, g2, be2)[0]

    sums, cnts, maxs = _pool(h3, batch3)
    return _head(sums, cnts, maxs, W2a, b2a, W2b, b2b, Wout, bout)


# speedup vs baseline: 5.0463x; 1.0191x over previous
import functools

import jax
import jax.numpy as jnp
from jax import lax
from jax.experimental import pallas as pl
from jax.experimental.pallas import tpu as pltpu
from jax.experimental.pallas import tpu_sc as plsc

N = 100000
E = 1600000
T = 12
F = 8
H = 48
EMB = 12
NSEG = 64
D = 48

NC, NS = 2, 16
NW = NC * NS
BLK = 128
NBUF = 6
NBLK_W = 396
NGRP = NBLK_W // NBUF
EP = NW * NBLK_W * BLK
ROWS_PER_TILE = 6256
ACC_ROWS = ROWS_PER_TILE * NS
ZR = 128

NB = 2000
GRID_N = N // NB
ENB = 1000
EGRID = N // ENB


def _make_sc_agg(with_cnt: bool):
    npass = 3 + (1 if with_cnt else 0)
    mesh = plsc.VectorSubcoreMesh(core_axis_name="c", subcore_axis_name="s",
                                  num_cores=NC, num_subcores=NS)

    @functools.partial(
        pl.kernel,
        out_type=jax.ShapeDtypeStruct((NC, npass, ACC_ROWS, 16), jnp.float32),
        mesh=mesh,
        scratch_types=[
            pltpu.VMEM((2, NBUF, BLK), jnp.int32),
            pltpu.VMEM((2, NBUF, BLK), jnp.int32),
            pltpu.VMEM((NBUF, BLK, 16), jnp.float32),
            pltpu.VMEM((BLK, 16), jnp.float32),
            pltpu.VMEM((ZR, 16), jnp.float32),
            pltpu.VMEM_SHARED((ACC_ROWS, 16), jnp.float32),
            pltpu.SemaphoreType.DMA((NBUF,)),
            pltpu.SemaphoreType.DMA((NBUF,)),
            pltpu.SemaphoreType.DMA((2, 2)),
        ],
        compiler_params=pltpu.CompilerParams(use_tc_tiling_on_sc=False),
    )
    def sc_agg(src_hbm, dst_hbm, t0, t1, t2, out_hbm,
               sbuf, dbuf, rows, ones_v, zbuf, acc, gsem, ssem, isem):
        c = lax.axis_index("c")
        s = lax.axis_index("s")
        w = c * NS + s
        base = s * ROWS_PER_TILE

        @pl.loop(0, BLK)
        def _init_ones(i):
            ones_v[i, :] = jnp.ones((16,), jnp.float32)

        @pl.loop(0, ZR)
        def _init_zeros(i):
            zbuf[i, :] = jnp.zeros((16,), jnp.float32)

        def zero_acc():
            nz = ROWS_PER_TILE // ZR
            rem = ROWS_PER_TILE - nz * ZR
            for ci in range(nz):
                pltpu.sync_copy(zbuf, acc.at[pl.ds(base + ci * ZR, ZR)])
            pltpu.sync_copy(zbuf.at[pl.ds(0, rem)],
                            acc.at[pl.ds(base + nz * ZR, rem)])

        def stage_idx(g, q, need_src):
            grp = pl.ds(g * NBUF, NBUF)
            if need_src:
                pltpu.async_copy(src_hbm.at[w, grp], sbuf.at[q], isem.at[q, 0])
            pltpu.async_copy(dst_hbm.at[w, grp], dbuf.at[q], isem.at[q, 1])

        def wait_idx(g, q, need_src):
            grp = pl.ds(g * NBUF, NBUF)
            if need_src:
                pltpu.make_async_copy(src_hbm.at[w, grp], sbuf.at[q],
                                      isem.at[q, 0]).wait()
            pltpu.make_async_copy(dst_hbm.at[w, grp], dbuf.at[q],
                                  isem.at[q, 1]).wait()

        def drain(p):
            plsc.subcore_barrier()
            pltpu.sync_copy(acc.at[pl.ds(base, ROWS_PER_TILE)],
                            out_hbm.at[c, p, pl.ds(base, ROWS_PER_TILE)])
            plsc.subcore_barrier()

        def feature_pass(table, p):
            zero_acc()
            plsc.subcore_barrier()
            stage_idx(0, 0, True)
            wait_idx(0, 0, True)
            stage_idx(1, 1, True)
            for b in range(NBUF):
                pltpu.async_copy(table.at[sbuf.at[0, b]], rows.at[b],
                                 gsem.at[b])

            @pl.loop(0, NGRP)
            def _grp(g):
                pb = lax.rem(g, 2)
                qb = lax.rem(g + 1, 2)
                for b in range(NBUF):
                    pltpu.make_async_copy(table.at[sbuf.at[pb, b]],
                                          rows.at[b], gsem.at[b]).wait()
                    pltpu.async_copy(rows.at[b], acc.at[dbuf.at[pb, b]],
                                     ssem.at[b], add=True)
                for b in range(NBUF):
                    pltpu.make_async_copy(rows.at[b], acc.at[dbuf.at[pb, b]],
                                          ssem.at[b]).wait()

                @pl.when(g + 2 < NGRP)
                def _():
                    stage_idx(g + 2, pb, True)

                @pl.when(g + 1 < NGRP)
                def _():
                    wait_idx(g + 1, qb, True)
                    for b in range(NBUF):
                        pltpu.async_copy(table.at[sbuf.at[qb, b]],
                                         rows.at[b], gsem.at[b])

            drain(p)

        def cnt_pass(p):
            zero_acc()
            plsc.subcore_barrier()
            stage_idx(0, 0, False)
            wait_idx(0, 0, False)
            stage_idx(1, 1, False)

            @pl.loop(0, NGRP)
            def _grp(g):
                pb = lax.rem(g, 2)
                qb = lax.rem(g + 1, 2)
                for b in range(NBUF):
                    pltpu.async_copy(ones_v, acc.at[dbuf.at[pb, b]],
                                     ssem.at[b], add=True)
                for b in range(NBUF):
                    pltpu.make_async_copy(ones_v, acc.at[dbuf.at[pb, b]],
                                          ssem.at[b]).wait()

                @pl.when(g + 2 < NGRP)
                def _():
                    stage_idx(g + 2, pb, False)

                @pl.when(g + 1 < NGRP)
                def _():
                    wait_idx(g + 1, qb, False)

            drain(p)

        feature_pass(t0, 0)
        feature_pass(t1, 1)
        feature_pass(t2, 2)
        if with_cnt:
            cnt_pass(3)

    return sc_agg


@functools.lru_cache(maxsize=None)
def _get_sc_agg(with_cnt: bool):
    return _make_sc_agg(with_cnt)


def _encoder_body(x_ref, xd_ref, xst_ref, emb_ref,
                  wir, wiz, win, whr, whz, whn,
                  bir, biz, bin_, bhr, bhz, bhn,
                  w1h, w1d, w1s, b1_ref,
                  h48, o0, o1, o2):
    f32 = jnp.float32
    xs = xst_ref[0, 0, :]
    oh = (xs[:, None] == lax.broadcasted_iota(jnp.int32, (1, 256), 1)
          ).astype(f32)
    st = jnp.dot(oh, emb_ref[...], preferred_element_type=f32)

    bf16 = jnp.bfloat16
    xall = x_ref[...].reshape(T * ENB, F).astype(bf16)
    gr = jnp.dot(xall, wir[...].astype(bf16), preferred_element_type=f32) + bir[...]
    gz = jnp.dot(xall, wiz[...].astype(bf16), preferred_element_type=f32) + biz[...]
    gn = jnp.dot(xall, win[...].astype(bf16), preferred_element_type=f32) + bin_[...]

    whr_b = whr[...].astype(bf16)
    whz_b = whz[...].astype(bf16)
    whn_b = whn[...].astype(bf16)
    h = jnp.zeros((ENB, H), f32)
    for t in range(T):
        row = slice(t * ENB, (t + 1) * ENB)
        hb = h.astype(bf16)
        h_r = jnp.dot(hb, whr_b, preferred_element_type=f32)
        h_z = jnp.dot(hb, whz_b, preferred_element_type=f32)
        h_n = jnp.dot(hb, whn_b, preferred_element_type=f32) + bhn[...]
        r = 0.5 * jnp.tanh(gr[row] + h_r) + 0.5
        z = 0.5 * jnp.tanh(gz[row] + h_z) + 0.5
        n = jnp.tanh(gn[row] + r * h_n)
        h = n + z * (h - n)

    h1 = (jnp.dot(h.astype(bf16), w1h[...].astype(bf16),
                  preferred_element_type=f32)
          + jnp.dot(xd_ref[...].astype(bf16), w1d[...].astype(bf16),
                    preferred_element_type=f32)
          + jnp.dot(st.astype(bf16), w1s[...].astype(bf16),
                    preferred_element_type=f32)
          + b1_ref[...])
    h1 = jnp.maximum(h1, 0.0)
    h48[...] = h1
    o0[...] = h1[:, 0:16]
    o1[...] = h1[:, 16:32]
    o2[...] = h1[:, 32:48]


def _encoder(x, xdims, xst3, emb, gw, w1h, w1d, w1s, b1):
    f32 = jnp.float32
    full = lambda shp: pl.BlockSpec(shp, lambda i: (0,) * len(shp))
    in_specs = [
        pl.BlockSpec((T, ENB, F), lambda i: (0, i, 0)),
        pl.BlockSpec((ENB, 2), lambda i: (i, 0)),
        pl.BlockSpec((1, 1, ENB), lambda i: (i, 0, 0)),
        full((256, EMB)),
    ]
    in_specs += [full(w.shape) for w in gw]
    in_specs += [full((H, H)), full((2, H)), full((EMB, H)), full((1, H))]
    out_shape = [
        jax.ShapeDtypeStruct((N, H), f32),
        jax.ShapeDtypeStruct((N, 16), f32),
        jax.ShapeDtypeStruct((N, 16), f32),
        jax.ShapeDtypeStruct((N, 16), f32),
    ]
    out_specs = [
        pl.BlockSpec((ENB, H), lambda i: (i, 0)),
        pl.BlockSpec((ENB, 16), lambda i: (i, 0)),
        pl.BlockSpec((ENB, 16), lambda i: (i, 0)),
        pl.BlockSpec((ENB, 16), lambda i: (i, 0)),
    ]
    return pl.pallas_call(
        _encoder_body, grid=(EGRID,), in_specs=in_specs,
        out_specs=out_specs, out_shape=out_shape,
    )(x, xdims, xst3, emb, *gw, w1h, w1d, w1s, b1)


def _transform_body(first, h_ref, a0, a1, a2, p0, p1, p2, *rest):
    f32 = jnp.float32
    if first:
        c0, c1, wl0, wl1, wl2, bl_ref, wr, g_ref, be_ref = rest[:9]
        out_refs = rest[9:]
        cnt = (c0[0, 0] + c1[0, 0])[:, 0:1]
    else:
        c0, wl0, wl1, wl2, bl_ref, wr, g_ref, be_ref = rest[:8]
        out_refs = rest[8:]
        cnt = c0[...]
    inv = 1.0 / jnp.maximum(cnt, 1.0)
    pre = (jnp.dot((a0[0, 0] + p0[0, 0]) * inv, wl0[...],
                   preferred_element_type=f32)
           + jnp.dot((a1[0, 0] + p1[0, 0]) * inv, wl1[...],
                     preferred_element_type=f32)
           + jnp.dot((a2[0, 0] + p2[0, 0]) * inv, wl2[...],
                     preferred_element_type=f32)
           + bl_ref[...]
           + jnp.dot(h_ref[...], wr[...], preferred_element_type=f32))
    mu = jnp.mean(pre, axis=-1, keepdims=True)
    var = jnp.mean((pre - mu) ** 2, axis=-1, keepdims=True)
    y = (pre - mu) * lax.rsqrt(var + 1e-5) * g_ref[...] + be_ref[...]
    y = jnp.maximum(y, 0.0)
    out_refs[0][...] = y
    if first:
        out_refs[1][...] = y[:, 0:16]
        out_refs[2][...] = y[:, 16:32]
        out_refs[3][...] = y[:, 32:48]
        out_refs[4][...] = cnt


def _part_spec(c, k):
    return pl.BlockSpec((1, 1, NB, 16),
                        lambda i, _c=c, _k=k: (_c, _k, i, 0))


def _transform(first, h, parts, cnt, wl, bl, wr, g, be):
    f32 = jnp.float32
    full = lambda shp: pl.BlockSpec(shp, lambda i: (0,) * len(shp))
    nb16 = pl.BlockSpec((NB, 16), lambda i: (i, 0))
    in_specs = [pl.BlockSpec((NB, H), lambda i: (i, 0))]
    in_specs += [_part_spec(0, 0), _part_spec(0, 1), _part_spec(0, 2),
                 _part_spec(1, 0), _part_spec(1, 1), _part_spec(1, 2)]
    if first:
        in_specs += [_part_spec(0, 3), _part_spec(1, 3)]
        cnt_args = (parts, parts)
    else:
        in_specs += [pl.BlockSpec((NB, 1), lambda i: (i, 0))]
        cnt_args = (cnt,)
    in_specs += [full((16, H))] * 3 + [full((1, H)), full((H, H)),
                                       full((1, H)), full((1, H))]
    out_shape = [jax.ShapeDtypeStruct((N, H), f32)]
    out_specs = [pl.BlockSpec((NB, H), lambda i: (i, 0))]
    if first:
        out_shape += [jax.ShapeDtypeStruct((N, 16), f32)] * 3
        out_shape += [jax.ShapeDtypeStruct((N, 1), f32)]
        out_specs += [nb16] * 3 + [pl.BlockSpec((NB, 1), lambda i: (i, 0))]
    wl0 = wl[:, 0:16].T
    wl1 = wl[:, 16:32].T
    wl2 = wl[:, 32:48].T
    return pl.pallas_call(
        functools.partial(_transform_body, first),
        grid=(GRID_N,), in_specs=in_specs, out_specs=out_specs,
        out_shape=out_shape,
    )(h, parts, parts, parts, parts, parts, parts, *cnt_args,
      wl0, wl1, wl2, bl.reshape(1, H), wr.T, g.reshape(1, H),
      be.reshape(1, H))


def _pool_body(h_ref, b_ref, sums, cnts, maxs):
    f32 = jnp.float32
    i = pl.program_id(0)

    @pl.when(i == 0)
    def _():
        sums[...] = jnp.zeros((NSEG, H), f32)
        cnts[...] = jnp.zeros((NSEG, 1), f32)
        maxs[...] = jnp.full((NSEG, H), -jnp.inf, f32)

    b = b_ref[0, 0, :]
    h = h_ref[...]
    oh = (b[:, None] == lax.broadcasted_iota(jnp.int32, (1, NSEG), 1)).astype(f32)
    sums[...] += lax.dot_general(oh, h, (((0,), (0,)), ((), ())),
                                 preferred_element_type=f32)
    cnts[...] += lax.dot_general(oh, jnp.ones((NB, 1), f32),
                                 (((0,), (0,)), ((), ())),
                                 preferred_element_type=f32)
    for s in range(NSEG):
        m = jnp.max(jnp.where(b[:, None] == s, h, -jnp.inf), axis=0)
        maxs[s:s + 1, :] = jnp.maximum(maxs[s:s + 1, :], m[None, :])


def _pool(h, batch3):
    f32 = jnp.float32
    full0 = lambda shp: pl.BlockSpec(shp, lambda i: (0,) * len(shp))
    return pl.pallas_call(
        _pool_body, grid=(GRID_N,),
        in_specs=[pl.BlockSpec((NB, H), lambda i: (i, 0)),
                  pl.BlockSpec((1, 1, NB), lambda i: (i, 0, 0))],
        out_specs=[full0((NSEG, H)), full0((NSEG, 1)), full0((NSEG, H))],
        out_shape=[jax.ShapeDtypeStruct((NSEG, H), f32),
                   jax.ShapeDtypeStruct((NSEG, 1), f32),
                   jax.ShapeDtypeStruct((NSEG, H), f32)],
    )(h, batch3)


def _head_body(sums, cnts, maxs, w2am, w2ax, b2a, w2b, b2b, wo, bo, out):
    f32 = jnp.float32
    gmean = sums[...] / jnp.maximum(cnts[...], 1.0)
    g = (jnp.dot(gmean, w2am[...], preferred_element_type=f32)
         + jnp.dot(maxs[...], w2ax[...], preferred_element_type=f32)
         + b2a[...])
    g = jnp.maximum(g, 0.0)
    g = jnp.maximum(jnp.dot(g, w2b[...], preferred_element_type=f32) + b2b[...], 0.0)
    out[...] = jnp.dot(g, wo[...], preferred_element_type=f32) + bo[...]


def _head(sums, cnts, maxs, W2a, b2a, W2b, b2b, Wout, bout):
    f32 = jnp.float32
    return pl.pallas_call(
        _head_body,
        out_shape=jax.ShapeDtypeStruct((NSEG, 1), f32),
    )(sums, cnts, maxs, W2a[:, :D].T, W2a[:, D:].T, b2a.reshape(1, 50),
      W2b.T, b2b.reshape(1, 50), Wout.T, bout.reshape(1, 1))


def kernel(x, xdims, xsttype, edge_index, batch, emb, W_ih, W_hh, b_ih, b_hh,
           W1, b1, Wl1, bl1, Wr1, g1, be1, Wl2, bl2, Wr2, g2, be2,
           W2a, b2a, W2b, b2b, Wout, bout):
    f32 = jnp.float32
    gw = [0.5 * W_ih[0:H].T, 0.5 * W_ih[H:2 * H].T, W_ih[2 * H:].T,
          0.5 * W_hh[0:H].T, 0.5 * W_hh[H:2 * H].T, W_hh[2 * H:].T,
          (0.5 * (b_ih[0:H] + b_hh[0:H])).reshape(1, H),
          (0.5 * (b_ih[H:2 * H] + b_hh[H:2 * H])).reshape(1, H),
          b_ih[2 * H:].reshape(1, H),
          jnp.zeros((1, H), jnp.float32), jnp.zeros((1, H), jnp.float32),
          b_hh[2 * H:].reshape(1, H)]
    w1h = W1[:, 0:H].T
    w1d = W1[:, H:H + 2].T
    w1s = W1[:, H + 2:].T
    xst3 = xsttype.astype(jnp.int32).reshape(EGRID, 1, ENB)
    batch3 = batch.astype(jnp.int32).reshape(GRID_N, 1, NB)

    src = edge_index[0].astype(jnp.int32)
    dst = edge_index[1].astype(jnp.int32)
    pad = EP - E
    src3 = jnp.concatenate([src, jnp.zeros((pad,), jnp.int32)]
                           ).reshape(NW, NBLK_W, BLK)
    pad_dst = N + (jnp.arange(pad, dtype=jnp.int32) % (ACC_ROWS - N))
    dst3 = jnp.concatenate([dst, pad_dst]).reshape(NW, NBLK_W, BLK)

    xT = jnp.transpose(x, (1, 0, 2))
    h48, t0, t1, t2 = _encoder(xT, xdims, xst3, emb, gw,
                               w1h, w1d, w1s, b1.reshape(1, H))

    p1 = _get_sc_agg(True)(src3, dst3, t0, t1, t2)
    h2, u0, u1, u2, cnt = _transform(True, h48, p1, None,
                                     Wl1, bl1, Wr1, g1, be1)

    p2 = _get_sc_agg(False)(src3, dst3, u0, u1, u2)
    h3 = _transform(False, h2, p2, cnt, Wl2, bl2, Wr2, g2, be2)[0]

    sums, cnts, maxs = _pool(h3, batch3)
    return _head(sums, cnts, maxs, W2a, b2a, W2b, b2b, Wout, bout)


# speedup vs baseline: 5.2048x; 1.0314x over previous
import functools

import jax
import jax.numpy as jnp
from jax import lax
from jax.experimental import pallas as pl
from jax.experimental.pallas import tpu as pltpu
from jax.experimental.pallas import tpu_sc as plsc

N = 100000
E = 1600000
T = 12
F = 8
H = 48
EMB = 12
NSEG = 64
D = 48

NC, NS = 2, 16
NW = NC * NS
BLK = 128
NBUF = 6
BLK_C0 = 504
BLK_C1 = 288
TOTBLK = NS * (BLK_C0 + BLK_C1)
EP = TOTBLK * BLK
ROWS_PER_TILE = 6256
ACC_ROWS = ROWS_PER_TILE * NS
ZR = 128

NB = 2000
GRID_N = N // NB
ENB = 1000
EGRID = N // ENB


def _make_sc_agg(with_cnt: bool):
    npass = 3 + (1 if with_cnt else 0)
    mesh = plsc.VectorSubcoreMesh(core_axis_name="c", subcore_axis_name="s",
                                  num_cores=NC, num_subcores=NS)

    @functools.partial(
        pl.kernel,
        out_type=jax.ShapeDtypeStruct((NC, npass, ACC_ROWS, 16), jnp.float32),
        mesh=mesh,
        scratch_types=[
            pltpu.VMEM((2, NBUF, BLK), jnp.int32),
            pltpu.VMEM((2, NBUF, BLK), jnp.int32),
            pltpu.VMEM((NBUF, BLK, 16), jnp.float32),
            pltpu.VMEM((BLK, 16), jnp.float32),
            pltpu.VMEM((ZR, 16), jnp.float32),
            pltpu.VMEM_SHARED((ACC_ROWS, 16), jnp.float32),
            pltpu.SemaphoreType.DMA((NBUF,)),
            pltpu.SemaphoreType.DMA((NBUF,)),
            pltpu.SemaphoreType.DMA((2, 2)),
        ],
        compiler_params=pltpu.CompilerParams(use_tc_tiling_on_sc=False),
    )
    def sc_agg(src_hbm, dst_hbm, t0, t1, t2, out_hbm,
               sbuf, dbuf, rows, ones_v, zbuf, acc, gsem, ssem, isem):
        c = lax.axis_index("c")
        s = lax.axis_index("s")
        base = s * ROWS_PER_TILE
        base_blk = jnp.where(c == 0, s * BLK_C0, NS * BLK_C0 + s * BLK_C1)
        ngrp = jnp.where(c == 0, BLK_C0 // NBUF, BLK_C1 // NBUF)

        @pl.loop(0, BLK)
        def _init_ones(i):
            ones_v[i, :] = jnp.ones((16,), jnp.float32)

        @pl.loop(0, ZR)
        def _init_zeros(i):
            zbuf[i, :] = jnp.zeros((16,), jnp.float32)

        def zero_acc():
            nz = ROWS_PER_TILE // ZR
            rem = ROWS_PER_TILE - nz * ZR
            for ci in range(nz):
                pltpu.sync_copy(zbuf, acc.at[pl.ds(base + ci * ZR, ZR)])
            pltpu.sync_copy(zbuf.at[pl.ds(0, rem)],
                            acc.at[pl.ds(base + nz * ZR, rem)])

        def stage_idx(g, q, need_src):
            grp = pl.ds(base_blk + g * NBUF, NBUF)
            if need_src:
                pltpu.async_copy(src_hbm.at[grp], sbuf.at[q], isem.at[q, 0])
            pltpu.async_copy(dst_hbm.at[grp], dbuf.at[q], isem.at[q, 1])

        def wait_idx(g, q, need_src):
            grp = pl.ds(base_blk + g * NBUF, NBUF)
            if need_src:
                pltpu.make_async_copy(src_hbm.at[grp], sbuf.at[q],
                                      isem.at[q, 0]).wait()
            pltpu.make_async_copy(dst_hbm.at[grp], dbuf.at[q],
                                  isem.at[q, 1]).wait()

        def drain(p):
            plsc.subcore_barrier()
            pltpu.sync_copy(acc.at[pl.ds(base, ROWS_PER_TILE)],
                            out_hbm.at[c, p, pl.ds(base, ROWS_PER_TILE)])
            plsc.subcore_barrier()

        def feature_pass(table, p):
            zero_acc()
            plsc.subcore_barrier()
            stage_idx(0, 0, True)
            wait_idx(0, 0, True)
            stage_idx(1, 1, True)
            for b in range(NBUF):
                pltpu.async_copy(table.at[sbuf.at[0, b]], rows.at[b],
                                 gsem.at[b])

            @pl.loop(0, ngrp)
            def _grp(g):
                pb = lax.rem(g, 2)
                qb = lax.rem(g + 1, 2)
                for b in range(NBUF):
                    pltpu.make_async_copy(table.at[sbuf.at[pb, b]],
                                          rows.at[b], gsem.at[b]).wait()
                    pltpu.async_copy(rows.at[b], acc.at[dbuf.at[pb, b]],
                                     ssem.at[b], add=True)
                for b in range(NBUF):
                    pltpu.make_async_copy(rows.at[b], acc.at[dbuf.at[pb, b]],
                                          ssem.at[b]).wait()

                @pl.when(g + 2 < ngrp)
                def _():
                    stage_idx(g + 2, pb, True)

                @pl.when(g + 1 < ngrp)
                def _():
                    wait_idx(g + 1, qb, True)
                    for b in range(NBUF):
                        pltpu.async_copy(table.at[sbuf.at[qb, b]],
                                         rows.at[b], gsem.at[b])

            drain(p)

        def cnt_pass(p):
            zero_acc()
            plsc.subcore_barrier()
            stage_idx(0, 0, False)
            wait_idx(0, 0, False)
            stage_idx(1, 1, False)

            @pl.loop(0, ngrp)
            def _grp(g):
                pb = lax.rem(g, 2)
                qb = lax.rem(g + 1, 2)
                for b in range(NBUF):
                    pltpu.async_copy(ones_v, acc.at[dbuf.at[pb, b]],
                                     ssem.at[b], add=True)
                for b in range(NBUF):
                    pltpu.make_async_copy(ones_v, acc.at[dbuf.at[pb, b]],
                                          ssem.at[b]).wait()

                @pl.when(g + 2 < ngrp)
                def _():
                    stage_idx(g + 2, pb, False)

                @pl.when(g + 1 < ngrp)
                def _():
                    wait_idx(g + 1, qb, False)

            drain(p)

        feature_pass(t0, 0)
        feature_pass(t1, 1)
        feature_pass(t2, 2)
        if with_cnt:
            cnt_pass(3)

    return sc_agg


@functools.lru_cache(maxsize=None)
def _get_sc_agg(with_cnt: bool):
    return _make_sc_agg(with_cnt)


def _encoder_body(x_ref, xd_ref, xst_ref, emb_ref,
                  wir, wiz, win, whr, whz, whn,
                  bir, biz, bin_, bhr, bhz, bhn,
                  w1h, w1d, w1s, b1_ref,
                  h48, o0, o1, o2):
    f32 = jnp.float32
    xs = xst_ref[0, 0, :]
    oh = (xs[:, None] == lax.broadcasted_iota(jnp.int32, (1, 256), 1)
          ).astype(f32)
    st = jnp.dot(oh, emb_ref[...], preferred_element_type=f32)

    bf16 = jnp.bfloat16
    xall = x_ref[...].reshape(T * ENB, F).astype(bf16)
    gr = jnp.dot(xall, wir[...].astype(bf16), preferred_element_type=f32) + bir[...]
    gz = jnp.dot(xall, wiz[...].astype(bf16), preferred_element_type=f32) + biz[...]
    gn = jnp.dot(xall, win[...].astype(bf16), preferred_element_type=f32) + bin_[...]

    whr_b = whr[...].astype(bf16)
    whz_b = whz[...].astype(bf16)
    whn_b = whn[...].astype(bf16)
    h = jnp.zeros((ENB, H), f32)
    for t in range(T):
        row = slice(t * ENB, (t + 1) * ENB)
        hb = h.astype(bf16)
        h_r = jnp.dot(hb, whr_b, preferred_element_type=f32)
        h_z = jnp.dot(hb, whz_b, preferred_element_type=f32)
        h_n = jnp.dot(hb, whn_b, preferred_element_type=f32) + bhn[...]
        r = 0.5 * jnp.tanh(gr[row] + h_r) + 0.5
        z = 0.5 * jnp.tanh(gz[row] + h_z) + 0.5
        n = jnp.tanh(gn[row] + r * h_n)
        h = n + z * (h - n)

    h1 = (jnp.dot(h.astype(bf16), w1h[...].astype(bf16),
                  preferred_element_type=f32)
          + jnp.dot(xd_ref[...].astype(bf16), w1d[...].astype(bf16),
                    preferred_element_type=f32)
          + jnp.dot(st.astype(bf16), w1s[...].astype(bf16),
                    preferred_element_type=f32)
          + b1_ref[...])
    h1 = jnp.maximum(h1, 0.0)
    h48[...] = h1
    o0[...] = h1[:, 0:16]
    o1[...] = h1[:, 16:32]
    o2[...] = h1[:, 32:48]


def _encoder(x, xdims, xst3, emb, gw, w1h, w1d, w1s, b1):
    f32 = jnp.float32
    full = lambda shp: pl.BlockSpec(shp, lambda i: (0,) * len(shp))
    in_specs = [
        pl.BlockSpec((T, ENB, F), lambda i: (0, i, 0)),
        pl.BlockSpec((ENB, 2), lambda i: (i, 0)),
        pl.BlockSpec((1, 1, ENB), lambda i: (i, 0, 0)),
        full((256, EMB)),
    ]
    in_specs += [full(w.shape) for w in gw]
    in_specs += [full((H, H)), full((2, H)), full((EMB, H)), full((1, H))]
    out_shape = [
        jax.ShapeDtypeStruct((N, H), f32),
        jax.ShapeDtypeStruct((N, 16), f32),
        jax.ShapeDtypeStruct((N, 16), f32),
        jax.ShapeDtypeStruct((N, 16), f32),
    ]
    out_specs = [
        pl.BlockSpec((ENB, H), lambda i: (i, 0)),
        pl.BlockSpec((ENB, 16), lambda i: (i, 0)),
        pl.BlockSpec((ENB, 16), lambda i: (i, 0)),
        pl.BlockSpec((ENB, 16), lambda i: (i, 0)),
    ]
    return pl.pallas_call(
        _encoder_body, grid=(EGRID,), in_specs=in_specs,
        out_specs=out_specs, out_shape=out_shape,
    )(x, xdims, xst3, emb, *gw, w1h, w1d, w1s, b1)


def _transform_body(first, h_ref, a0, a1, a2, p0, p1, p2, *rest):
    f32 = jnp.float32
    if first:
        c0, c1, wl0, wl1, wl2, bl_ref, wr, g_ref, be_ref = rest[:9]
        out_refs = rest[9:]
        cnt = (c0[0, 0] + c1[0, 0])[:, 0:1]
    else:
        c0, wl0, wl1, wl2, bl_ref, wr, g_ref, be_ref = rest[:8]
        out_refs = rest[8:]
        cnt = c0[...]
    inv = 1.0 / jnp.maximum(cnt, 1.0)
    pre = (jnp.dot((a0[0, 0] + p0[0, 0]) * inv, wl0[...],
                   preferred_element_type=f32)
           + jnp.dot((a1[0, 0] + p1[0, 0]) * inv, wl1[...],
                     preferred_element_type=f32)
           + jnp.dot((a2[0, 0] + p2[0, 0]) * inv, wl2[...],
                     preferred_element_type=f32)
           + bl_ref[...]
           + jnp.dot(h_ref[...], wr[...], preferred_element_type=f32))
    mu = jnp.mean(pre, axis=-1, keepdims=True)
    var = jnp.mean((pre - mu) ** 2, axis=-1, keepdims=True)
    y = (pre - mu) * lax.rsqrt(var + 1e-5) * g_ref[...] + be_ref[...]
    y = jnp.maximum(y, 0.0)
    out_refs[0][...] = y
    if first:
        out_refs[1][...] = y[:, 0:16]
        out_refs[2][...] = y[:, 16:32]
        out_refs[3][...] = y[:, 32:48]
        out_refs[4][...] = cnt


def _part_spec(c, k):
    return pl.BlockSpec((1, 1, NB, 16),
                        lambda i, _c=c, _k=k: (_c, _k, i, 0))


def _transform(first, h, parts, cnt, wl, bl, wr, g, be):
    f32 = jnp.float32
    full = lambda shp: pl.BlockSpec(shp, lambda i: (0,) * len(shp))
    nb16 = pl.BlockSpec((NB, 16), lambda i: (i, 0))
    in_specs = [pl.BlockSpec((NB, H), lambda i: (i, 0))]
    in_specs += [_part_spec(0, 0), _part_spec(0, 1), _part_spec(0, 2),
                 _part_spec(1, 0), _part_spec(1, 1), _part_spec(1, 2)]
    if first:
        in_specs += [_part_spec(0, 3), _part_spec(1, 3)]
        cnt_args = (parts, parts)
    else:
        in_specs += [pl.BlockSpec((NB, 1), lambda i: (i, 0))]
        cnt_args = (cnt,)
    in_specs += [full((16, H))] * 3 + [full((1, H)), full((H, H)),
                                       full((1, H)), full((1, H))]
    out_shape = [jax.ShapeDtypeStruct((N, H), f32)]
    out_specs = [pl.BlockSpec((NB, H), lambda i: (i, 0))]
    if first:
        out_shape += [jax.ShapeDtypeStruct((N, 16), f32)] * 3
        out_shape += [jax.ShapeDtypeStruct((N, 1), f32)]
        out_specs += [nb16] * 3 + [pl.BlockSpec((NB, 1), lambda i: (i, 0))]
    wl0 = wl[:, 0:16].T
    wl1 = wl[:, 16:32].T
    wl2 = wl[:, 32:48].T
    return pl.pallas_call(
        functools.partial(_transform_body, first),
        grid=(GRID_N,), in_specs=in_specs, out_specs=out_specs,
        out_shape=out_shape,
    )(h, parts, parts, parts, parts, parts, parts, *cnt_args,
      wl0, wl1, wl2, bl.reshape(1, H), wr.T, g.reshape(1, H),
      be.reshape(1, H))


def _pool_body(h_ref, b_ref, sums, cnts, maxs):
    f32 = jnp.float32
    i = pl.program_id(0)

    @pl.when(i == 0)
    def _():
        sums[...] = jnp.zeros((NSEG, H), f32)
        cnts[...] = jnp.zeros((NSEG, 1), f32)
        maxs[...] = jnp.full((NSEG, H), -jnp.inf, f32)

    b = b_ref[0, 0, :]
    h = h_ref[...]
    oh = (b[:, None] == lax.broadcasted_iota(jnp.int32, (1, NSEG), 1)).astype(f32)
    sums[...] += lax.dot_general(oh, h, (((0,), (0,)), ((), ())),
                                 preferred_element_type=f32)
    cnts[...] += lax.dot_general(oh, jnp.ones((NB, 1), f32),
                                 (((0,), (0,)), ((), ())),
                                 preferred_element_type=f32)
    for s in range(NSEG):
        m = jnp.max(jnp.where(b[:, None] == s, h, -jnp.inf), axis=0)
        maxs[s:s + 1, :] = jnp.maximum(maxs[s:s + 1, :], m[None, :])


def _pool(h, batch3):
    f32 = jnp.float32
    full0 = lambda shp: pl.BlockSpec(shp, lambda i: (0,) * len(shp))
    return pl.pallas_call(
        _pool_body, grid=(GRID_N,),
        in_specs=[pl.BlockSpec((NB, H), lambda i: (i, 0)),
                  pl.BlockSpec((1, 1, NB), lambda i: (i, 0, 0))],
        out_specs=[full0((NSEG, H)), full0((NSEG, 1)), full0((NSEG, H))],
        out_shape=[jax.ShapeDtypeStruct((NSEG, H), f32),
                   jax.ShapeDtypeStruct((NSEG, 1), f32),
                   jax.ShapeDtypeStruct((NSEG, H), f32)],
    )(h, batch3)


def _head_body(sums, cnts, maxs, w2am, w2ax, b2a, w2b, b2b, wo, bo, out):
    f32 = jnp.float32
    gmean = sums[...] / jnp.maximum(cnts[...], 1.0)
    g = (jnp.dot(gmean, w2am[...], preferred_element_type=f32)
         + jnp.dot(maxs[...], w2ax[...], preferred_element_type=f32)
         + b2a[...])
    g = jnp.maximum(g, 0.0)
    g = jnp.maximum(jnp.dot(g, w2b[...], preferred_element_type=f32) + b2b[...], 0.0)
    out[...] = jnp.dot(g, wo[...], preferred_element_type=f32) + bo[...]


def _head(sums, cnts, maxs, W2a, b2a, W2b, b2b, Wout, bout):
    f32 = jnp.float32
    return pl.pallas_call(
        _head_body,
        out_shape=jax.ShapeDtypeStruct((NSEG, 1), f32),
    )(sums, cnts, maxs, W2a[:, :D].T, W2a[:, D:].T, b2a.reshape(1, 50),
      W2b.T, b2b.reshape(1, 50), Wout.T, bout.reshape(1, 1))


def kernel(x, xdims, xsttype, edge_index, batch, emb, W_ih, W_hh, b_ih, b_hh,
           W1, b1, Wl1, bl1, Wr1, g1, be1, Wl2, bl2, Wr2, g2, be2,
           W2a, b2a, W2b, b2b, Wout, bout):
    f32 = jnp.float32
    gw = [0.5 * W_ih[0:H].T, 0.5 * W_ih[H:2 * H].T, W_ih[2 * H:].T,
          0.5 * W_hh[0:H].T, 0.5 * W_hh[H:2 * H].T, W_hh[2 * H:].T,
          (0.5 * (b_ih[0:H] + b_hh[0:H])).reshape(1, H),
          (0.5 * (b_ih[H:2 * H] + b_hh[H:2 * H])).reshape(1, H),
          b_ih[2 * H:].reshape(1, H),
          jnp.zeros((1, H), jnp.float32), jnp.zeros((1, H), jnp.float32),
          b_hh[2 * H:].reshape(1, H)]
    w1h = W1[:, 0:H].T
    w1d = W1[:, H:H + 2].T
    w1s = W1[:, H + 2:].T
    xst3 = xsttype.astype(jnp.int32).reshape(EGRID, 1, ENB)
    batch3 = batch.astype(jnp.int32).reshape(GRID_N, 1, NB)

    src = edge_index[0].astype(jnp.int32)
    dst = edge_index[1].astype(jnp.int32)
    pad = EP - E
    src3 = jnp.concatenate([src, jnp.zeros((pad,), jnp.int32)]
                           ).reshape(TOTBLK, BLK)
    pad_dst = N + (jnp.arange(pad, dtype=jnp.int32) % (ACC_ROWS - N))
    dst3 = jnp.concatenate([dst, pad_dst]).reshape(TOTBLK, BLK)

    xT = jnp.transpose(x, (1, 0, 2))
    h48, t0, t1, t2 = _encoder(xT, xdims, xst3, emb, gw,
                               w1h, w1d, w1s, b1.reshape(1, H))

    p1 = _get_sc_agg(True)(src3, dst3, t0, t1, t2)
    h2, u0, u1, u2, cnt = _transform(True, h48, p1, None,
                                     Wl1, bl1, Wr1, g1, be1)

    p2 = _get_sc_agg(False)(src3, dst3, u0, u1, u2)
    h3 = _transform(False, h2, p2, cnt, Wl2, bl2, Wr2, g2, be2)[0]

    sums, cnts, maxs = _pool(h3, batch3)
    return _head(sums, cnts, maxs, W2a, b2a, W2b, b2b, Wout, bout)
